# baseline (speedup 1.0000x reference)
import functools

import jax
import jax.numpy as jnp
from jax import lax
from jax.experimental import pallas as pl
from jax.experimental.pallas import tpu as pltpu
from jax.experimental.pallas import tpu_sc as plsc

_PREFIX = 32768
_S = 64
_CACHE_COLS = _PREFIX + _S * 16
_OUT_COLS = _PREFIX + _S
_NEG = jnp.finfo(jnp.float32).min
_NW = 32
_RPW = _S // _NW
_EYE_PAD = 128


@functools.partial(
    pl.kernel,
    out_type=jax.ShapeDtypeStruct((_S, _CACHE_COLS), jnp.bool_),
    mesh=plsc.VectorSubcoreMesh(core_axis_name="c", subcore_axis_name="s"),
    scratch_types=[
        pltpu.VMEM((_RPW,), jnp.int32),
        pltpu.VMEM((_RPW, _CACHE_COLS), jnp.bool_),
        pltpu.VMEM((_RPW, _EYE_PAD), jnp.bool_),
        pltpu.SemaphoreType.DMA,
        pltpu.SemaphoreType.DMA,
    ],
)
def _sc_gather(table_hbm, idx_hbm, eye_hbm, out_hbm, idx_v, rows_v, eye_v, sem0, sem1):
    wid = lax.axis_index("s") * 2 + lax.axis_index("c")
    base = wid * _RPW
    eye_cp = pltpu.make_async_copy(eye_hbm.at[wid], eye_v, sem1)
    eye_cp.start()
    pltpu.sync_copy(idx_hbm.at[wid], idx_v)
    pltpu.async_copy(table_hbm.at[idx_v], rows_v, sem0).wait()
    pltpu.sync_copy(
        rows_v.at[:, pl.ds(0, _PREFIX)],
        out_hbm.at[pl.ds(base, _RPW), pl.ds(0, _PREFIX)],
    )
    eye_cp.wait()
    pltpu.sync_copy(
        eye_v, out_hbm.at[pl.ds(base, _RPW), pl.ds(_PREFIX, _EYE_PAD)]
    )


def _convert_body(g_ref, out_ref):
    zero = jnp.float32(0.0)
    neg = jnp.float32(_NEG)
    out_ref[...] = jnp.where(g_ref[:, :_OUT_COLS], zero, neg)


def kernel(parent_indices, tree_mask_cache, eye_block):
    cache = tree_mask_cache.reshape(_S, _CACHE_COLS)
    eye = eye_block.reshape(_S, _S)
    eye_slab = jnp.pad(eye, ((0, 0), (0, _EYE_PAD - _S))).reshape(
        _NW, _RPW, _EYE_PAD
    )
    idx = parent_indices.reshape(_NW, _RPW)

    gathered = _sc_gather(cache, idx, eye_slab)

    out = pl.pallas_call(
        _convert_body,
        grid=(2,),
        in_specs=[pl.BlockSpec((32, _CACHE_COLS), lambda i: (i, 0))],
        out_specs=pl.BlockSpec((32, _OUT_COLS), lambda i: (i, 0)),
        out_shape=jax.ShapeDtypeStruct((_S, _OUT_COLS), jnp.float32),
    )(gathered)
    return out.reshape(1, 1, _S, _OUT_COLS)


# speedup vs baseline: 1.0289x; 1.0289x over previous
import functools

import jax
import jax.numpy as jnp
from jax import lax
from jax.experimental import pallas as pl
from jax.experimental.pallas import tpu as pltpu
from jax.experimental.pallas import tpu_sc as plsc

_PREFIX = 32768
_S = 64
_CACHE_COLS = _PREFIX + _S * 16
_OUT_COLS = _PREFIX + _S
_NEG = jnp.finfo(jnp.float32).min
_NW = 32
_RPW = _S // _NW
_BLK = 32


@functools.partial(
    pl.kernel,
    out_type=jax.ShapeDtypeStruct((_S, _CACHE_COLS), jnp.bool_),
    mesh=plsc.VectorSubcoreMesh(core_axis_name="c", subcore_axis_name="s"),
    scratch_types=[
        pltpu.VMEM((_RPW,), jnp.int32),
        pltpu.VMEM((_RPW, _CACHE_COLS), jnp.bool_),
        pltpu.SemaphoreType.DMA,
    ],
)
def _sc_gather(table_hbm, idx_hbm, out_hbm, idx_v, rows_v, sem):
    wid = lax.axis_index("s") * 2 + lax.axis_index("c")
    base = wid * _RPW
    pltpu.sync_copy(idx_hbm.at[wid], idx_v)
    pltpu.async_copy(table_hbm.at[idx_v], rows_v, sem).wait()
    pltpu.sync_copy(rows_v, out_hbm.at[pl.ds(base, _RPW)])


def _convert_body(g_ref, out_ref):
    zero = jnp.float32(0.0)
    neg = jnp.float32(_NEG)
    row0 = _BLK * pl.program_id(0)
    ri = lax.broadcasted_iota(jnp.int32, (_BLK, _S), 0) + row0
    ci = lax.broadcasted_iota(jnp.int32, (_BLK, _S), 1)
    mask = jnp.concatenate([g_ref[:, :_PREFIX], ri == ci], axis=1)
    out_ref[...] = jnp.where(mask, zero, neg)


def kernel(parent_indices, tree_mask_cache, eye_block):
    del eye_block
    cache = tree_mask_cache.reshape(_S, _CACHE_COLS)
    idx = parent_indices.reshape(_NW, _RPW)

    gathered = _sc_gather(cache, idx)

    out = pl.pallas_call(
        _convert_body,
        grid=(_S // _BLK,),
        in_specs=[pl.BlockSpec((_BLK, _CACHE_COLS), lambda i: (i, 0))],
        out_specs=pl.BlockSpec((_BLK, _OUT_COLS), lambda i: (i, 0)),
        out_shape=jax.ShapeDtypeStruct((_S, _OUT_COLS), jnp.float32),
    )(gathered)
    return out.reshape(1, 1, _S, _OUT_COLS)
